# baseline (speedup 1.0000x reference)
import functools

import jax
import jax.numpy as jnp
from jax.experimental import pallas as pl
from jax.experimental.pallas import tpu as pltpu

_EDGE_FEAT_DIMS = (5, 6, 2)
_FEAT_OFFSETS = (0, 5, 11)
_K = 128


def _fused_kernel(attr_ref, table_ref, out_ref, *, n_feat):
    attr = attr_ref[...]
    tile_e = attr.shape[0]
    lane = jax.lax.broadcasted_iota(jnp.int32, (tile_e, _K), 1)

    hit = None
    for f in range(n_feat):
        a = attr[:, f : f + 1]
        a = jnp.clip(a, 0, _EDGE_FEAT_DIMS[f] - 1) + _FEAT_OFFSETS[f]
        m = lane == a
        hit = m if hit is None else (hit | m)
    multi_hot = hit.astype(jnp.float32)

    out_ref[...] = jnp.dot(
        multi_hot, table_ref[...], preferred_element_type=jnp.float32
    )


def kernel(edge_attr, packed_table):
    E, F = edge_attr.shape
    W, PH = packed_table.shape
    rows_per_copy = 16
    pack = W // rows_per_copy
    nhid = PH // pack

    edge_attr = edge_attr.astype(jnp.int32)
    base = packed_table[:rows_per_copy, :nhid]
    table = jnp.zeros((_K, nhid), jnp.float32).at[:rows_per_copy, :].set(base)

    tile_e = 8192
    while tile_e > 8 and E % tile_e:
        tile_e //= 2
    pad = (-E) % tile_e
    if pad:
        edge_attr = jnp.pad(edge_attr, ((0, pad), (0, 0)))
    e_pad = E + pad
    n = e_pad // tile_e

    if n % 2 == 0:
        grid = (2, n // 2)
        in_index = lambda c, i: (c * (n // 2) + i, 0)
        tab_index = lambda c, i: (0, 0)
        semantics = ("parallel", "arbitrary")
    else:
        grid = (n,)
        in_index = lambda i: (i, 0)
        tab_index = lambda i: (0, 0)
        semantics = ("parallel",)

    out = pl.pallas_call(
        functools.partial(_fused_kernel, n_feat=F),
        out_shape=jax.ShapeDtypeStruct((e_pad, nhid), jnp.float32),
        grid=grid,
        in_specs=[
            pl.BlockSpec((tile_e, F), in_index),
            pl.BlockSpec((_K, nhid), tab_index),
        ],
        out_specs=pl.BlockSpec((tile_e, nhid), in_index),
        compiler_params=pltpu.CompilerParams(
            dimension_semantics=semantics,
        ),
    )(edge_attr, table)

    return out[:E] if pad else out


# speedup vs baseline: 1.0327x; 1.0327x over previous
import functools

import jax
import jax.numpy as jnp
from jax.experimental import pallas as pl
from jax.experimental.pallas import tpu as pltpu

_EDGE_FEAT_DIMS = (5, 6, 2)
_FEAT_OFFSETS = (0, 5, 11)
_K = 128


def _fused_kernel(attr_ref, table_ref, out_ref, *, n_feat):
    attr = attr_ref[...]
    tile_e = attr.shape[0]
    lane = jax.lax.broadcasted_iota(jnp.int32, (tile_e, _K), 1)

    hit = None
    for f in range(n_feat):
        a = attr[:, f : f + 1]
        a = jnp.clip(a, 0, _EDGE_FEAT_DIMS[f] - 1) + _FEAT_OFFSETS[f]
        m = lane == a
        hit = m if hit is None else (hit | m)
    multi_hot = hit.astype(jnp.float32)

    out_ref[...] = jnp.dot(
        multi_hot, table_ref[...], preferred_element_type=jnp.float32
    )


def kernel(edge_attr, packed_table):
    E, F = edge_attr.shape
    W, PH = packed_table.shape
    rows_per_copy = 16
    pack = W // rows_per_copy
    nhid = PH // pack

    edge_attr = edge_attr.astype(jnp.int32)
    base = packed_table[:rows_per_copy, :nhid]
    table = jnp.zeros((_K, nhid), jnp.float32).at[:rows_per_copy, :].set(base)

    tile_e = 16384
    while tile_e > 8 and E % tile_e:
        tile_e //= 2
    pad = (-E) % tile_e
    if pad:
        edge_attr = jnp.pad(edge_attr, ((0, pad), (0, 0)))
    e_pad = E + pad
    n = e_pad // tile_e

    if n % 2 == 0:
        grid = (2, n // 2)
        in_index = lambda c, i: (c * (n // 2) + i, 0)
        tab_index = lambda c, i: (0, 0)
        semantics = ("parallel", "arbitrary")
    else:
        grid = (n,)
        in_index = lambda i: (i, 0)
        tab_index = lambda i: (0, 0)
        semantics = ("parallel",)

    out = pl.pallas_call(
        functools.partial(_fused_kernel, n_feat=F),
        out_shape=jax.ShapeDtypeStruct((e_pad, nhid), jnp.float32),
        grid=grid,
        in_specs=[
            pl.BlockSpec((tile_e, F), in_index),
            pl.BlockSpec((_K, nhid), tab_index),
        ],
        out_specs=pl.BlockSpec((tile_e, nhid), in_index),
        compiler_params=pltpu.CompilerParams(
            dimension_semantics=semantics,
        ),
    )(edge_attr, table)

    return out[:E] if pad else out
